# baseline (speedup 1.0000x reference)
import functools

import jax
import jax.numpy as jnp
from jax import lax
from jax.experimental import pallas as pl
from jax.experimental.pallas import tpu as pltpu
from jax.experimental.pallas import tpu_sc as plsc

T = 8
COMPANIES = 100000
POSITIONS = 100000
D = 64
B = 16384

NC = 2
NS = 16
L = 16
NW = NC * NS
BPW = B // NW
CHUNK = 128
NCHUNK = BPW // CHUNK


def _sc_body(c_hbm, p_hbm, te_hbm, com_hbm, pos_hbm,
             out_com_hbm, out_pos_hbm,
             c_v, p_v, te_v, idx_com_v, idx_pos_v,
             rows_com_v, rows_pos_v, sem_idx, sem_rows):
    wid = lax.axis_index("s") * NC + lax.axis_index("c")
    base = wid * BPW

    cp_c = pltpu.make_async_copy(c_hbm.at[pl.ds(base, BPW)], c_v, sem_idx)
    cp_p = pltpu.make_async_copy(p_hbm.at[pl.ds(base, BPW)], p_v, sem_idx)
    cp_t = pltpu.make_async_copy(te_hbm.at[pl.ds(base, BPW)], te_v, sem_idx)
    cp_c.start()
    cp_p.start()
    cp_t.start()
    cp_c.wait()
    cp_p.wait()
    cp_t.wait()

    for i in range(BPW // L):
        j = i // (CHUNK // L)
        k = (i % (CHUNK // L)) * L
        te = te_v[pl.ds(i * L, L)]
        idx_com_v[j, pl.ds(k, L)] = te * COMPANIES + c_v[pl.ds(i * L, L)]
        idx_pos_v[j, pl.ds(k, L)] = te * POSITIONS + p_v[pl.ds(i * L, L)]

    copies = []
    for j in range(NCHUNK):
        copies.append(pltpu.make_async_copy(
            com_hbm.at[idx_com_v.at[j]],
            rows_com_v.at[pl.ds(j * CHUNK, CHUNK)], sem_rows))
        copies.append(pltpu.make_async_copy(
            pos_hbm.at[idx_pos_v.at[j]],
            rows_pos_v.at[pl.ds(j * CHUNK, CHUNK)], sem_rows))
    for cp in copies:
        cp.start()
    for cp in copies:
        cp.wait()

    out_c = pltpu.make_async_copy(rows_com_v, out_com_hbm.at[pl.ds(base, BPW)],
                                  sem_rows)
    out_p = pltpu.make_async_copy(rows_pos_v, out_pos_hbm.at[pl.ds(base, BPW)],
                                  sem_rows)
    out_c.start()
    out_p.start()
    out_c.wait()
    out_p.wait()


@jax.jit
def _sc_gather(c, p, t_e, com_flat, pos_flat):
    mesh = plsc.VectorSubcoreMesh(core_axis_name="c", subcore_axis_name="s",
                                  num_cores=NC, num_subcores=NS)
    return pl.kernel(
        _sc_body,
        out_type=(jax.ShapeDtypeStruct((B, D), jnp.float32),
                  jax.ShapeDtypeStruct((B, D), jnp.float32)),
        mesh=mesh,
        compiler_params=pltpu.CompilerParams(use_tc_tiling_on_sc=False),
        scratch_types=[
            pltpu.VMEM((BPW,), jnp.int32),
            pltpu.VMEM((BPW,), jnp.int32),
            pltpu.VMEM((BPW,), jnp.int32),
            pltpu.VMEM((NCHUNK, CHUNK), jnp.int32),
            pltpu.VMEM((NCHUNK, CHUNK), jnp.int32),
            pltpu.VMEM((BPW, D), jnp.float32),
            pltpu.VMEM((BPW, D), jnp.float32),
            pltpu.SemaphoreType.DMA,
            pltpu.SemaphoreType.DMA,
        ],
    )(c, p, t_e, com_flat, pos_flat)


def kernel(c, p, t_s, t_e, com_embs, pos_embs):
    del t_s
    com_flat = com_embs.reshape(T * COMPANIES, D)
    pos_flat = pos_embs.reshape(T * POSITIONS, D)
    return _sc_gather(c, p, t_e, com_flat, pos_flat)


# speedup vs baseline: 3.2796x; 3.2796x over previous
import functools

import jax
import jax.numpy as jnp
from jax import lax
from jax.experimental import pallas as pl
from jax.experimental.pallas import tpu as pltpu
from jax.experimental.pallas import tpu_sc as plsc

T = 8
COMPANIES = 100000
POSITIONS = 100000
D = 64
B = 16384

NC = 2
NS = 16
NW = NC * NS
NJ = 782
JPW = 25
CAP = 2048
HROWS = 256
RING = 16
INFLIGHT = 8


def _splat(s):
    return lax.broadcast_in_dim(jnp.int32(s) if isinstance(s, int) else s,
                                (16,), ())


def _iota():
    return lax.iota(jnp.int32, 16)


def _sc_body(c_hbm, p_hbm, te_hbm, com_hbm, pos_hbm,
             out_com_hbm, out_pos_hbm,
             c_v, p_v, te_v, tjc_v, blc_v, tjp_v, blp_v,
             slab0_v, slab1_v, stage_c_v, stage_p_v,
             sem_in, sem_s0, sem_s1, sem_oc, sem_op):
    wid = lax.axis_index("s") * NC + lax.axis_index("c")
    wid_s = _splat(wid)

    cp_c = pltpu.make_async_copy(c_hbm, c_v, sem_in)
    cp_p = pltpu.make_async_copy(p_hbm, p_v, sem_in)
    cp_t = pltpu.make_async_copy(te_hbm, te_v, sem_in)
    cp_c.start(); cp_p.start(); cp_t.start()
    cp_c.wait(); cp_p.wait(); cp_t.wait()

    def pbody(i, carry):
        cc_v, cp_v2 = carry
        sl = pl.ds(i * 16, 16)
        cv = c_v[sl]
        pv = p_v[sl]
        tev = te_v[sl]
        bv = _splat(i * 16) + _iota()

        jc = lax.shift_right_logical(cv, 7)
        mc = (jc & 31) == wid_s
        rank = plsc.cumsum(mc.astype(jnp.int32)) - 1
        dst = cc_v + rank
        plsc.store_scatter(tjc_v, [dst], jc * 8 + tev, mask=mc)
        plsc.store_scatter(blc_v, [dst], bv * 128 + (cv & 127), mask=mc)
        cc_v = cc_v + plsc.all_reduce_population_count(mc)

        jp = lax.shift_right_logical(pv, 7)
        mp = (jp & 31) == wid_s
        rankp = plsc.cumsum(mp.astype(jnp.int32)) - 1
        dstp = cp_v2 + rankp
        plsc.store_scatter(tjp_v, [dstp], jp * 8 + tev, mask=mp)
        plsc.store_scatter(blp_v, [dstp], bv * 128 + (pv & 127), mask=mp)
        cp_v2 = cp_v2 + plsc.all_reduce_population_count(mp)
        return (cc_v, cp_v2)

    zeros = _splat(0)
    cc_v, cp_v2 = lax.fori_loop(0, B // 16, pbody, (zeros, zeros))
    cnt_c = jnp.max(cc_v)
    cnt_p = jnp.max(cp_v2)

    def slab_wait(sem, slot_ref):
        pltpu.make_async_copy(
            com_hbm.at[pl.ds(0, HROWS), pl.ds(0, 128)], slot_ref, sem).wait()

    def slab_fetch(tbl_hbm, h, j, slot_ref, sem):
        pltpu.make_async_copy(
            tbl_hbm.at[pl.ds(h * HROWS, HROWS), pl.ds(j * 128, 128)],
            slot_ref, sem).start()

    def scan(slot_ref, tj_list, bl_list, cnt, j2h, out_ref, sem_out, oc,
             stage_v):
        cnt_s = _splat(cnt)
        j2h_s = _splat(j2h)
        nk = lax.shift_right_logical(cnt + 15, 4)

        def kbody(k, oc_):
            sl = pl.ds(k * 16, 16)
            tjv = tj_list[sl]
            blv = bl_list[sl]
            lane_ok = (_splat(k * 16) + _iota()) < cnt_s
            m = (lax.shift_right_logical(tjv, 2) == j2h_s) & lane_ok

            def wcond(carry):
                m_, _ = carry
                return jnp.max(m_.astype(jnp.int32)) > 0

            def wbody(carry):
                m_, o_ = carry
                ffs = plsc.all_reduce_ffs(m_)
                sel = _iota() == ffs
                bl_s = jnp.max(jnp.where(sel, blv, 0))
                tj_s = jnp.max(jnp.where(sel, tjv, 0))
                b = lax.shift_right_logical(bl_s, 7)
                l = bl_s & 127
                te_loc = tj_s & 3
                slot = o_ & (RING - 1)
                for kk in range(4):
                    rvec = _splat(te_loc * 64 + kk * 16) + _iota()
                    vals = plsc.load_gather(slot_ref, [rvec, _splat(l)])
                    stage_v[pl.ds(slot * 64 + kk * 16, 16)] = vals

                @pl.when(o_ >= INFLIGHT)
                def _():
                    pltpu.make_async_copy(
                        stage_v.at[pl.ds(0, 64)],
                        out_ref.at[pl.ds(0, 64)], sem_out).wait()

                pltpu.make_async_copy(
                    stage_v.at[pl.ds(slot * 64, 64)],
                    out_ref.at[pl.ds(b * 64, 64)], sem_out).start()
                return (m_ & (~sel), o_ + 1)

            _, oc_ = lax.while_loop(wcond, wbody, (m, oc_))
            return oc_

        return lax.fori_loop(0, nk, kbody, oc)

    slab_fetch(com_hbm, 0, wid, slab0_v, sem_s0)
    slab_fetch(com_hbm, 1, wid, slab1_v, sem_s1)

    def jbody(jj, carry):
        oc_c, oc_p = carry
        j = wid + jj * 32
        jn = j + 32
        valid = j < NJ
        validn = jn < NJ

        @pl.when(valid)
        def _():
            slab_wait(sem_s0, slab0_v)
        oc_c = scan(slab0_v, tjc_v, blc_v, cnt_c, j * 2 + 0,
                    out_com_hbm, sem_oc, oc_c, stage_c_v)
        @pl.when(valid)
        def _():
            slab_fetch(pos_hbm, 0, j, slab0_v, sem_s0)

        @pl.when(valid)
        def _():
            slab_wait(sem_s1, slab1_v)
        oc_c = scan(slab1_v, tjc_v, blc_v, cnt_c, j * 2 + 1,
                    out_com_hbm, sem_oc, oc_c, stage_c_v)
        @pl.when(valid)
        def _():
            slab_fetch(pos_hbm, 1, j, slab1_v, sem_s1)

        @pl.when(valid)
        def _():
            slab_wait(sem_s0, slab0_v)
        oc_p = scan(slab0_v, tjp_v, blp_v, cnt_p, j * 2 + 0,
                    out_pos_hbm, sem_op, oc_p, stage_p_v)
        @pl.when(validn)
        def _():
            slab_fetch(com_hbm, 0, jn, slab0_v, sem_s0)

        @pl.when(valid)
        def _():
            slab_wait(sem_s1, slab1_v)
        oc_p = scan(slab1_v, tjp_v, blp_v, cnt_p, j * 2 + 1,
                    out_pos_hbm, sem_op, oc_p, stage_p_v)
        @pl.when(validn)
        def _():
            slab_fetch(com_hbm, 1, jn, slab1_v, sem_s1)

        return (oc_c, oc_p)

    oc_c, oc_p = lax.fori_loop(0, JPW, jbody,
                               (jnp.int32(0), jnp.int32(0)))

    def drain(n, out_ref, sem, stage_v):
        def db(i, _):
            pltpu.make_async_copy(
                stage_v.at[pl.ds(0, 64)],
                out_ref.at[pl.ds(0, 64)], sem).wait()
            return 0
        lax.fori_loop(0, n, db, 0)

    drain(jnp.minimum(oc_c, INFLIGHT), out_com_hbm, sem_oc, stage_c_v)
    drain(jnp.minimum(oc_p, INFLIGHT), out_pos_hbm, sem_op, stage_p_v)


@jax.jit
def _sc_gather(c, p, t_e, com2d, pos2d):
    mesh = plsc.VectorSubcoreMesh(core_axis_name="c", subcore_axis_name="s",
                                  num_cores=NC, num_subcores=NS)
    return pl.kernel(
        _sc_body,
        out_type=(jax.ShapeDtypeStruct((B * D,), jnp.float32),
                  jax.ShapeDtypeStruct((B * D,), jnp.float32)),
        mesh=mesh,
        compiler_params=pltpu.CompilerParams(use_tc_tiling_on_sc=True,
                                             disable_bounds_checks=True,
                                             needs_layout_passes=False),
        scratch_types=[
            pltpu.VMEM((B,), jnp.int32),
            pltpu.VMEM((B,), jnp.int32),
            pltpu.VMEM((B,), jnp.int32),
            pltpu.VMEM((CAP,), jnp.int32),
            pltpu.VMEM((CAP,), jnp.int32),
            pltpu.VMEM((CAP,), jnp.int32),
            pltpu.VMEM((CAP,), jnp.int32),
            pltpu.VMEM((HROWS, 128), jnp.float32),
            pltpu.VMEM((HROWS, 128), jnp.float32),
            pltpu.VMEM((RING * D,), jnp.float32),
            pltpu.VMEM((RING * D,), jnp.float32),
            pltpu.SemaphoreType.DMA,
            pltpu.SemaphoreType.DMA,
            pltpu.SemaphoreType.DMA,
            pltpu.SemaphoreType.DMA,
            pltpu.SemaphoreType.DMA,
        ],
    )(c, p, t_e, com2d, pos2d)


def kernel(c, p, t_s, t_e, com_embs, pos_embs):
    del t_s
    com2d = com_embs.transpose(0, 2, 1).reshape(T * D, COMPANIES)
    pos2d = pos_embs.transpose(0, 2, 1).reshape(T * D, POSITIONS)
    out_com, out_pos = _sc_gather(c, p, t_e, com2d, pos2d)
    return (out_com.reshape(B, D), out_pos.reshape(B, D))


# speedup vs baseline: 3.2971x; 1.0053x over previous
import functools

import jax
import jax.numpy as jnp
from jax import lax
from jax.experimental import pallas as pl
from jax.experimental.pallas import tpu as pltpu
from jax.experimental.pallas import tpu_sc as plsc

T = 8
COMPANIES = 100000
POSITIONS = 100000
D = 64
B = 16384

NC = 2
NS = 16
NW = NC * NS
NJ = 782
JPW = 25
CAP = 2048
HROWS = 256
RING = 32
INFLIGHT = 24


def _splat(s):
    return lax.broadcast_in_dim(jnp.int32(s) if isinstance(s, int) else s,
                                (16,), ())


def _iota():
    return lax.iota(jnp.int32, 16)


def _sc_body(c_hbm, p_hbm, te_hbm, com_hbm, pos_hbm,
             out_com_hbm, out_pos_hbm,
             c_v, p_v, te_v, tjc_v, blc_v, tjp_v, blp_v,
             slab0_v, slab1_v, stage_c_v, stage_p_v,
             sem_in, sem_s0, sem_s1, sem_oc, sem_op):
    wid = lax.axis_index("s") * NC + lax.axis_index("c")
    wid_s = _splat(wid)

    pltpu.make_async_copy(
        com_hbm.at[pl.ds(0, HROWS), pl.ds(wid * 128, 128)],
        slab0_v, sem_s0).start()
    pltpu.make_async_copy(
        com_hbm.at[pl.ds(HROWS, HROWS), pl.ds(wid * 128, 128)],
        slab1_v, sem_s1).start()

    cp_c = pltpu.make_async_copy(c_hbm, c_v, sem_in)
    cp_p = pltpu.make_async_copy(p_hbm, p_v, sem_in)
    cp_t = pltpu.make_async_copy(te_hbm, te_v, sem_in)
    cp_c.start(); cp_p.start(); cp_t.start()
    cp_c.wait(); cp_p.wait(); cp_t.wait()

    def pbody(i, carry):
        cc_v, cp_v2 = carry
        sl = pl.ds(i * 16, 16)
        cv = c_v[sl]
        pv = p_v[sl]
        tev = te_v[sl]
        bv = _splat(i * 16) + _iota()

        jc = lax.shift_right_logical(cv, 7)
        mc = (jc & 31) == wid_s
        rank = plsc.cumsum(mc.astype(jnp.int32)) - 1
        dst = cc_v + rank
        plsc.store_scatter(tjc_v, [dst], jc * 8 + tev, mask=mc)
        plsc.store_scatter(blc_v, [dst], bv * 128 + (cv & 127), mask=mc)
        cc_v = cc_v + plsc.all_reduce_population_count(mc)

        jp = lax.shift_right_logical(pv, 7)
        mp = (jp & 31) == wid_s
        rankp = plsc.cumsum(mp.astype(jnp.int32)) - 1
        dstp = cp_v2 + rankp
        plsc.store_scatter(tjp_v, [dstp], jp * 8 + tev, mask=mp)
        plsc.store_scatter(blp_v, [dstp], bv * 128 + (pv & 127), mask=mp)
        cp_v2 = cp_v2 + plsc.all_reduce_population_count(mp)
        return (cc_v, cp_v2)

    zeros = _splat(0)
    cc_v, cp_v2 = lax.fori_loop(0, B // 16, pbody, (zeros, zeros))
    cnt_c = jnp.max(cc_v)
    cnt_p = jnp.max(cp_v2)

    def slab_wait(sem, slot_ref):
        pltpu.make_async_copy(
            com_hbm.at[pl.ds(0, HROWS), pl.ds(0, 128)], slot_ref, sem).wait()

    def slab_fetch(tbl_hbm, h, j, slot_ref, sem):
        pltpu.make_async_copy(
            tbl_hbm.at[pl.ds(h * HROWS, HROWS), pl.ds(j * 128, 128)],
            slot_ref, sem).start()

    def scan(slot_ref, tj_list, bl_list, cnt, j2h, out_ref, sem_out, oc,
             stage_v):
        cnt_s = _splat(cnt)
        j2h_s = _splat(j2h)
        nk = lax.shift_right_logical(cnt + 15, 4)

        def kbody(k, oc_):
            sl = pl.ds(k * 16, 16)
            tjv = tj_list[sl]
            blv = bl_list[sl]
            lane_ok = (_splat(k * 16) + _iota()) < cnt_s
            m = (lax.shift_right_logical(tjv, 2) == j2h_s) & lane_ok

            def wcond(carry):
                m_, _ = carry
                return jnp.max(m_.astype(jnp.int32)) > 0

            def wbody(carry):
                m_, o_ = carry
                ffs = plsc.all_reduce_ffs(m_)
                sel = _iota() == ffs
                bl_s = jnp.max(jnp.where(sel, blv, 0))
                tj_s = jnp.max(jnp.where(sel, tjv, 0))
                b = lax.shift_right_logical(bl_s, 7)
                l = bl_s & 127
                te_loc = tj_s & 3
                slot = o_ & (RING - 1)
                for kk in range(4):
                    rvec = _splat(te_loc * 64 + kk * 16) + _iota()
                    vals = plsc.load_gather(slot_ref, [rvec, _splat(l)])
                    stage_v[pl.ds(slot * 64 + kk * 16, 16)] = vals

                @pl.when(o_ >= INFLIGHT)
                def _():
                    pltpu.make_async_copy(
                        stage_v.at[pl.ds(0, 64)],
                        out_ref.at[pl.ds(0, 64)], sem_out).wait()

                pltpu.make_async_copy(
                    stage_v.at[pl.ds(slot * 64, 64)],
                    out_ref.at[pl.ds(b * 64, 64)], sem_out).start()
                return (m_ & (~sel), o_ + 1)

            _, oc_ = lax.while_loop(wcond, wbody, (m, oc_))
            return oc_

        return lax.fori_loop(0, nk, kbody, oc)

    def jbody(jj, carry):
        oc_c, oc_p = carry
        j = wid + jj * 32
        jn = j + 32
        valid = j < NJ
        validn = jn < NJ

        @pl.when(valid)
        def _():
            slab_wait(sem_s0, slab0_v)
        oc_c = scan(slab0_v, tjc_v, blc_v, cnt_c, j * 2 + 0,
                    out_com_hbm, sem_oc, oc_c, stage_c_v)
        @pl.when(valid)
        def _():
            slab_fetch(pos_hbm, 0, j, slab0_v, sem_s0)

        @pl.when(valid)
        def _():
            slab_wait(sem_s1, slab1_v)
        oc_c = scan(slab1_v, tjc_v, blc_v, cnt_c, j * 2 + 1,
                    out_com_hbm, sem_oc, oc_c, stage_c_v)
        @pl.when(valid)
        def _():
            slab_fetch(pos_hbm, 1, j, slab1_v, sem_s1)

        @pl.when(valid)
        def _():
            slab_wait(sem_s0, slab0_v)
        oc_p = scan(slab0_v, tjp_v, blp_v, cnt_p, j * 2 + 0,
                    out_pos_hbm, sem_op, oc_p, stage_p_v)
        @pl.when(validn)
        def _():
            slab_fetch(com_hbm, 0, jn, slab0_v, sem_s0)

        @pl.when(valid)
        def _():
            slab_wait(sem_s1, slab1_v)
        oc_p = scan(slab1_v, tjp_v, blp_v, cnt_p, j * 2 + 1,
                    out_pos_hbm, sem_op, oc_p, stage_p_v)
        @pl.when(validn)
        def _():
            slab_fetch(com_hbm, 1, jn, slab1_v, sem_s1)

        return (oc_c, oc_p)

    oc_c, oc_p = lax.fori_loop(0, JPW, jbody,
                               (jnp.int32(0), jnp.int32(0)))

    def drain(n, out_ref, sem, stage_v):
        def db(i, _):
            pltpu.make_async_copy(
                stage_v.at[pl.ds(0, 64)],
                out_ref.at[pl.ds(0, 64)], sem).wait()
            return 0
        lax.fori_loop(0, n, db, 0)

    drain(jnp.minimum(oc_c, INFLIGHT), out_com_hbm, sem_oc, stage_c_v)
    drain(jnp.minimum(oc_p, INFLIGHT), out_pos_hbm, sem_op, stage_p_v)


@jax.jit
def _sc_gather(c, p, t_e, com2d, pos2d):
    mesh = plsc.VectorSubcoreMesh(core_axis_name="c", subcore_axis_name="s",
                                  num_cores=NC, num_subcores=NS)
    return pl.kernel(
        _sc_body,
        out_type=(jax.ShapeDtypeStruct((B * D,), jnp.float32),
                  jax.ShapeDtypeStruct((B * D,), jnp.float32)),
        mesh=mesh,
        compiler_params=pltpu.CompilerParams(use_tc_tiling_on_sc=True,
                                             disable_bounds_checks=True,
                                             needs_layout_passes=False),
        scratch_types=[
            pltpu.VMEM((B,), jnp.int32),
            pltpu.VMEM((B,), jnp.int32),
            pltpu.VMEM((B,), jnp.int32),
            pltpu.VMEM((CAP,), jnp.int32),
            pltpu.VMEM((CAP,), jnp.int32),
            pltpu.VMEM((CAP,), jnp.int32),
            pltpu.VMEM((CAP,), jnp.int32),
            pltpu.VMEM((HROWS, 128), jnp.float32),
            pltpu.VMEM((HROWS, 128), jnp.float32),
            pltpu.VMEM((RING * D,), jnp.float32),
            pltpu.VMEM((RING * D,), jnp.float32),
            pltpu.SemaphoreType.DMA,
            pltpu.SemaphoreType.DMA,
            pltpu.SemaphoreType.DMA,
            pltpu.SemaphoreType.DMA,
            pltpu.SemaphoreType.DMA,
        ],
    )(c, p, t_e, com2d, pos2d)


def kernel(c, p, t_s, t_e, com_embs, pos_embs):
    del t_s
    com2d = com_embs.transpose(0, 2, 1).reshape(T * D, COMPANIES)
    pos2d = pos_embs.transpose(0, 2, 1).reshape(T * D, POSITIONS)
    out_com, out_pos = _sc_gather(c, p, t_e, com2d, pos2d)
    return (out_com.reshape(B, D), out_pos.reshape(B, D))
